# baseline (speedup 1.0000x reference)
import functools

import jax
import jax.numpy as jnp
from jax import lax
from jax.experimental import pallas as pl
from jax.experimental.pallas import tpu as pltpu
from jax.experimental.pallas import tpu_sc as plsc

N = 10000
E = 320000
D = 128
DE = 16
H = 4
HD = D // H
SCALE = 1.0 / (HD ** 0.5)

BN = 1000
BE = 512

NC = 2
NS = 16
NW = NC * NS
EPW = E // NW
CH = 80
KJ = EPW // CH
NP = 10240
NPS = NP // NS
ZR = 128


def _gelu(t):
    return 0.5 * t * (1.0 + lax.erf(t * (2.0 ** -0.5)))


def _ln_body(x_ref, g_ref, b_ref, o_ref):
    x = x_ref[...]
    m = jnp.mean(x, axis=-1, keepdims=True)
    v = jnp.mean((x - m) * (x - m), axis=-1, keepdims=True)
    o_ref[...] = (x - m) / jnp.sqrt(v + 1e-5) * g_ref[...] + b_ref[...]


def _ln1(x, g, b):
    return pl.pallas_call(
        _ln_body,
        grid=(N // BN,),
        in_specs=[
            pl.BlockSpec((BN, D), lambda i: (i, 0)),
            pl.BlockSpec((1, D), lambda i: (0, 0)),
            pl.BlockSpec((1, D), lambda i: (0, 0)),
        ],
        out_specs=pl.BlockSpec((BN, D), lambda i: (i, 0)),
        out_shape=jax.ShapeDtypeStruct((N, D), jnp.float32),
    )(x, g[None, :], b[None, :])


def _edge_body(hs_ref, hd_ref, ee_ref,
               wm1a, wm1b, wm2, wg1d, wg1s, wg1e, wg2p,
               wa1d, wa1s, wa1e, wa2p, r_ref,
               bm1, bm2, bg1, bg2p, ba1, ba2p, mask16,
               u_ref, ex_ref):
    bf16 = jnp.bfloat16
    hs = hs_ref[...].astype(bf16)
    hd = hd_ref[...].astype(bf16)
    ee = ee_ref[...].astype(bf16)
    f32 = jnp.float32
    dot = lambda a, b: jax.lax.dot_general(
        a.astype(bf16), b.astype(bf16), (((1,), (0,)), ((), ())),
        preferred_element_type=f32)
    t_m = _gelu(dot(hs, wm1a[...]) + dot(ee, wm1b[...]) + bm1[...])
    msg = dot(t_m, wm2[...]) + bm2[...]
    t_g = _gelu(dot(hd, wg1d[...]) + dot(hs, wg1s[...]) + dot(ee, wg1e[...])
                + bg1[...])
    g16 = jax.nn.sigmoid(dot(t_g, wg2p[...]) + bg2p[...])
    t_a = _gelu(dot(hd, wa1d[...]) + dot(hs, wa1s[...]) + dot(ee, wa1e[...])
                + ba1[...])
    s16 = (dot(t_a, wa2p[...]) + ba2p[...]) * SCALE
    ex16 = jnp.exp(s16) * mask16[...]
    r = r_ref[...]
    u_ref[...] = msg * dot(g16, r) * dot(ex16, r)
    ex_ref[...] = ex16


def _edge_mlp(hs, hd, ee, wm1a, wm1b, wm2, wg1d, wg1s, wg1e, wg2p,
              wa1d, wa1s, wa1e, wa2p, rmat,
              bm1, bm2, bg1, bg2p, ba1, ba2p, mask16):
    full = lambda s: pl.BlockSpec(s, lambda i: tuple(0 for _ in s))
    return pl.pallas_call(
        _edge_body,
        grid=(E // BE,),
        in_specs=[
            pl.BlockSpec((BE, D), lambda i: (i, 0)),
            pl.BlockSpec((BE, D), lambda i: (i, 0)),
            pl.BlockSpec((BE, DE), lambda i: (i, 0)),
            full((D, D)), full((DE, D)), full((D, D)),
            full((D, D)), full((D, D)), full((DE, D)), full((D, 16)),
            full((D, D)), full((D, D)), full((DE, D)), full((D, 16)),
            full((16, D)),
            full((1, D)), full((1, D)), full((1, D)), full((1, 16)),
            full((1, D)), full((1, 16)), full((1, 16)),
        ],
        out_specs=[
            pl.BlockSpec((BE, D), lambda i: (i, 0)),
            pl.BlockSpec((BE, 16), lambda i: (i, 0)),
        ],
        out_shape=[
            jax.ShapeDtypeStruct((E, D), jnp.float32),
            jax.ShapeDtypeStruct((E, 16), jnp.float32),
        ],
    )(hs, hd, ee, wm1a, wm1b, wm2, wg1d, wg1s, wg1e, wg2p,
      wa1d, wa1s, wa1e, wa2p, rmat, bm1, bm2, bg1, bg2p, ba1, ba2p, mask16)


def _final_body(x_ref, h_ref, ap_ref, rdn_ref,
                wself, wagg, wf1, wf2,
                bsa, bf1, bf2, g2, b2, o_ref):
    f32 = jnp.float32
    dot = lambda a, b: jax.lax.dot_general(
        a, b, (((1,), (0,)), ((), ())), preferred_element_type=f32)
    agg = (ap_ref[0] + ap_ref[1]) * rdn_ref[...]
    upd = dot(h_ref[...], wself[...]) + dot(agg, wagg[...]) + bsa[...]
    o1 = x_ref[...] + upd
    m = jnp.mean(o1, axis=-1, keepdims=True)
    v = jnp.mean((o1 - m) * (o1 - m), axis=-1, keepdims=True)
    ln = (o1 - m) / jnp.sqrt(v + 1e-5) * g2[...] + b2[...]
    f = dot(_gelu(dot(ln, wf1[...]) + bf1[...]), wf2[...]) + bf2[...]
    o_ref[...] = o1 + f


def _final(x, h, agg_p, rdn_n, wself, wagg, wf1, wf2,
           bsa, bf1, bf2, g2, b2):
    full = lambda s: pl.BlockSpec(s, lambda i: tuple(0 for _ in s))
    return pl.pallas_call(
        _final_body,
        grid=(N // BN,),
        in_specs=[
            pl.BlockSpec((BN, D), lambda i: (i, 0)),
            pl.BlockSpec((BN, D), lambda i: (i, 0)),
            pl.BlockSpec((2, BN, D), lambda i: (0, i, 0)),
            pl.BlockSpec((BN, D), lambda i: (i, 0)),
            full((D, D)), full((D, D)), full((D, 2 * D)), full((2 * D, D)),
            full((1, D)), full((1, 2 * D)), full((1, D)),
            full((1, D)), full((1, D)),
        ],
        out_specs=pl.BlockSpec((BN, D), lambda i: (i, 0)),
        out_shape=jax.ShapeDtypeStruct((N, D), jnp.float32),
    )(x, h, agg_p, rdn_n, wself, wagg, wf1, wf2,
      bsa, bf1, bf2, g2, b2)


def _sc_gather_body(h_hbm, src3, dst3, hs_out, hd_out,
                    idx_s, idx_d, bufa, bufb, sema, semb):
    c = lax.axis_index("c")
    s = lax.axis_index("s")
    wid = c * NS + s
    base = wid * EPW
    pltpu.sync_copy(src3.at[wid], idx_s)
    pltpu.sync_copy(dst3.at[wid], idx_d)

    def _step(j, _):
        cpa = pltpu.async_copy(h_hbm.at[idx_s.at[j]], bufa, sema)
        cpb = pltpu.async_copy(h_hbm.at[idx_d.at[j]], bufb, semb)
        cpa.wait()
        pltpu.sync_copy(bufa, hs_out.at[pl.ds(base + j * CH, CH), :])
        cpb.wait()
        pltpu.sync_copy(bufb, hd_out.at[pl.ds(base + j * CH, CH), :])
        return 0
    lax.fori_loop(0, KJ, _step, 0)


def _sc_gather(h, src3, dst3):
    f32 = jnp.float32
    return pl.kernel(
        _sc_gather_body,
        out_type=[jax.ShapeDtypeStruct((E, D), f32),
                  jax.ShapeDtypeStruct((E, D), f32)],
        mesh=plsc.VectorSubcoreMesh(core_axis_name="c", subcore_axis_name="s"),
        scratch_types=[
            pltpu.VMEM((KJ, CH), jnp.int32),
            pltpu.VMEM((KJ, CH), jnp.int32),
            pltpu.VMEM((CH, D), f32),
            pltpu.VMEM((CH, D), f32),
            pltpu.SemaphoreType.DMA,
            pltpu.SemaphoreType.DMA,
        ],
    )(h, src3, dst3)


def _sc_agg_body(dst3, u_hbm, agg_out, idx_v, ub, agg_s):
    c = lax.axis_index("c")
    s = lax.axis_index("s")
    wid = c * NS + s
    base = wid * EPW
    z16 = jnp.zeros((16,), jnp.float32)

    def _zrow(r, _):
        for l in range(D // 16):
            ub[r, pl.ds(l * 16, 16)] = z16
        return 0
    lax.fori_loop(0, CH, _zrow, 0)

    def _zdma(t, _):
        pltpu.sync_copy(ub, agg_s.at[pl.ds(s * NPS + t * CH, CH), :])
        return 0
    lax.fori_loop(0, NPS // CH, _zdma, 0)
    plsc.subcore_barrier()

    pltpu.sync_copy(dst3.at[wid], idx_v)

    def _step(j, _):
        pltpu.sync_copy(u_hbm.at[pl.ds(base + j * CH, CH), :], ub)
        pltpu.sync_copy(ub, agg_s.at[idx_v.at[j]], add=True)
        return 0
    lax.fori_loop(0, KJ, _step, 0)
    plsc.subcore_barrier()

    pltpu.sync_copy(agg_s.at[pl.ds(s * NPS, NPS), :],
                    agg_out.at[c, pl.ds(s * NPS, NPS), :])


def _sc_agg(dst3, u):
    f32 = jnp.float32
    return pl.kernel(
        _sc_agg_body,
        out_type=jax.ShapeDtypeStruct((NC, NP, D), f32),
        mesh=plsc.VectorSubcoreMesh(core_axis_name="c", subcore_axis_name="s"),
        scratch_types=[
            pltpu.VMEM((KJ, CH), jnp.int32),
            pltpu.VMEM((CH, D), f32),
            pltpu.VMEM_SHARED((NP, D), f32),
        ],
    )(dst3, u)


def _sc_dnacc_body(dst3, ex_hbm, dn_out, idx_v, exb, pb, dn_s):
    c = lax.axis_index("c")
    s = lax.axis_index("s")
    wid = c * NS + s
    base = wid * EPW
    z16 = jnp.zeros((16,), jnp.float32)

    def _zrow(r, _):
        for l in range(D // 16):
            pb[r, pl.ds(l * 16, 16)] = z16
        return 0
    lax.fori_loop(0, CH, _zrow, 0)

    def _zdma(t, _):
        pltpu.sync_copy(pb, dn_s.at[pl.ds(s * NPS + t * CH, CH), :])
        return 0
    lax.fori_loop(0, NPS // CH, _zdma, 0)
    plsc.subcore_barrier()

    pltpu.sync_copy(dst3.at[wid], idx_v)

    def _step(j, _):
        pltpu.sync_copy(ex_hbm.at[pl.ds(base + j * CH, CH), :], exb)

        def _row(r, _):
            pb[r, pl.ds(0, 16)] = exb[r, :]
            return 0
        lax.fori_loop(0, CH, _row, 0)
        pltpu.sync_copy(pb, dn_s.at[idx_v.at[j]], add=True)
        return 0
    lax.fori_loop(0, KJ, _step, 0)
    plsc.subcore_barrier()

    pltpu.sync_copy(dn_s.at[pl.ds(s * NPS, NPS), :],
                    dn_out.at[c, pl.ds(s * NPS, NPS), :])


def _sc_dnacc(dst3, ex):
    f32 = jnp.float32
    return pl.kernel(
        _sc_dnacc_body,
        out_type=jax.ShapeDtypeStruct((NC, NP, D), f32),
        mesh=plsc.VectorSubcoreMesh(core_axis_name="c", subcore_axis_name="s"),
        scratch_types=[
            pltpu.VMEM((KJ, CH), jnp.int32),
            pltpu.VMEM((CH, 16), f32),
            pltpu.VMEM((CH, D), f32),
            pltpu.VMEM_SHARED((NP, D), f32),
        ],
    )(dst3, ex)


BNP = 1280


def _rdnrep_body(d_ref, r_ref, o_ref):
    dn16 = d_ref[0, :, :16] + d_ref[1, :, :16]
    rdn = 1.0 / jnp.clip(dn16, 1e-12, None)
    o_ref[...] = jax.lax.dot_general(
        rdn, r_ref[...], (((1,), (0,)), ((), ())),
        preferred_element_type=jnp.float32)


def _rdnrep(dn_pp, rmat):
    return pl.pallas_call(
        _rdnrep_body,
        grid=(NP // BNP,),
        in_specs=[pl.BlockSpec((NC, BNP, D), lambda i: (0, i, 0)),
                  pl.BlockSpec((16, D), lambda i: (0, 0))],
        out_specs=pl.BlockSpec((BNP, D), lambda i: (i, 0)),
        out_shape=jax.ShapeDtypeStruct((NP, D), jnp.float32),
    )(dn_pp, rmat)


def _sc_er_body(u_hbm, rdnrep, dst3, er_out, idx_v, ub, gb, rdn_s, sem):
    c = lax.axis_index("c")
    s = lax.axis_index("s")
    wid = c * NS + s
    base = wid * EPW

    @pl.when(s == 0)
    def _():
        pltpu.sync_copy(rdnrep, rdn_s)
    plsc.subcore_barrier()

    pltpu.sync_copy(dst3.at[wid], idx_v)

    def _step(j, _):
        pltpu.sync_copy(u_hbm.at[pl.ds(base + j * CH, CH), :], ub)
        pltpu.async_copy(rdn_s.at[idx_v.at[j]], gb, sem).wait()

        def _row(r, _):
            for l in range(D // 16):
                sl = pl.ds(l * 16, 16)
                ub[r, sl] = ub[r, sl] * gb[r, sl]
            return 0
        lax.fori_loop(0, CH, _row, 0)
        pltpu.sync_copy(ub, er_out.at[pl.ds(base + j * CH, CH), :])
        return 0
    lax.fori_loop(0, KJ, _step, 0)


def _sc_er(u, rdnrep, dst3):
    f32 = jnp.float32
    return pl.kernel(
        _sc_er_body,
        out_type=jax.ShapeDtypeStruct((E, D), f32),
        mesh=plsc.VectorSubcoreMesh(core_axis_name="c", subcore_axis_name="s"),
        scratch_types=[
            pltpu.VMEM((KJ, CH), jnp.int32),
            pltpu.VMEM((CH, D), f32),
            pltpu.VMEM((CH, D), f32),
            pltpu.VMEM_SHARED((NP, D), f32),
            pltpu.SemaphoreType.DMA,
        ],
    )(u, rdnrep, dst3)


def kernel(x, edge_src, edge_dst, edge_emb, ln1_g, ln1_b, ln2_g, ln2_b,
           W_self, b_self, W_msg1, b_msg1, W_msg2, b_msg2,
           W_attn1, b_attn1, W_attn2, b_attn2,
           W_gate1, b_gate1, W_gate2, b_gate2,
           W_agg, b_agg, W_ffn1, b_ffn1, W_ffn2, b_ffn2):
    f32 = jnp.float32
    wm1a, wm1b = W_msg1[:D], W_msg1[D:]
    wg1d, wg1s, wg1e = W_gate1[:D], W_gate1[D:2 * D], W_gate1[2 * D:]
    wa1d, wa1s, wa1e = W_attn1[:D], W_attn1[D:2 * D], W_attn1[2 * D:]
    wg2p = jnp.pad(W_gate2, ((0, 0), (0, 16 - H)))
    wa2p = jnp.pad(W_attn2, ((0, 0), (0, 16 - H)))
    bg2p = jnp.pad(b_gate2, (0, 16 - H))[None, :]
    ba2p = jnp.pad(b_attn2, (0, 16 - H))[None, :]
    rmat = (jnp.arange(D)[None, :] // HD == jnp.arange(16)[:, None]).astype(f32)
    mask16 = (jnp.arange(16) < H).astype(f32)[None, :]
    bsa = (b_self + b_agg)[None, :]

    h = _ln1(x, ln1_g, ln1_b)

    src3 = edge_src.reshape(NW, KJ, CH)
    dst3 = edge_dst.reshape(NW, KJ, CH)
    hs, hd = _sc_gather(h, src3, dst3)

    u, ex = _edge_mlp(hs, hd, edge_emb,
                      wm1a, wm1b, W_msg2, wg1d, wg1s, wg1e, wg2p,
                      wa1d, wa1s, wa1e, wa2p, rmat,
                      b_msg1[None, :], b_msg2[None, :], b_gate1[None, :],
                      bg2p, b_attn1[None, :], ba2p, mask16)

    agg_pp = _sc_agg(dst3, u)
    agg_p = agg_pp[:, :N]

    dn_pp = _sc_dnacc(dst3, ex)
    rdn_rep = _rdnrep(dn_pp, rmat)

    er = _sc_er(u, rdn_rep, dst3)

    out = _final(x, h, agg_p, rdn_rep[:N], W_self, W_agg, W_ffn1, W_ffn2,
                 bsa, b_ffn1[None, :], b_ffn2[None, :],
                 ln2_g[None, :], ln2_b[None, :])
    return (out, er)


# speedup vs baseline: 1.0179x; 1.0179x over previous
import functools

import jax
import jax.numpy as jnp
from jax import lax
from jax.experimental import pallas as pl
from jax.experimental.pallas import tpu as pltpu
from jax.experimental.pallas import tpu_sc as plsc

N = 10000
E = 320000
D = 128
DE = 16
H = 4
HD = D // H
SCALE = 1.0 / (HD ** 0.5)

BN = 1000
BE = 512

NC = 2
NS = 16
NW = NC * NS
EPW = E // NW
CH = 80
KJ = EPW // CH
NP = 10240
NPS = NP // NS
ZR = 128


def _gelu(t):
    return 0.5 * t * (1.0 + lax.erf(t * (2.0 ** -0.5)))


def _ln_body(x_ref, g_ref, b_ref, o_ref):
    x = x_ref[...]
    m = jnp.mean(x, axis=-1, keepdims=True)
    v = jnp.mean((x - m) * (x - m), axis=-1, keepdims=True)
    o_ref[...] = (x - m) / jnp.sqrt(v + 1e-5) * g_ref[...] + b_ref[...]


def _ln1(x, g, b):
    return pl.pallas_call(
        _ln_body,
        grid=(N // BN,),
        in_specs=[
            pl.BlockSpec((BN, D), lambda i: (i, 0)),
            pl.BlockSpec((1, D), lambda i: (0, 0)),
            pl.BlockSpec((1, D), lambda i: (0, 0)),
        ],
        out_specs=pl.BlockSpec((BN, D), lambda i: (i, 0)),
        out_shape=jax.ShapeDtypeStruct((N, D), jnp.float32),
    )(x, g[None, :], b[None, :])


def _edge_body(hs_ref, hd_ref, ee_ref,
               wm1a, wm1b, wm2, wg1d, wg1s, wg1e, wg2p,
               wa1d, wa1s, wa1e, wa2p, r_ref,
               bm1, bm2, bg1, bg2p, ba1, ba2p, mask16,
               u_ref, ex_ref):
    hs = hs_ref[...]
    hd = hd_ref[...]
    ee = ee_ref[...]
    f32 = jnp.float32
    dot = lambda a, b: jax.lax.dot_general(
        a, b, (((1,), (0,)), ((), ())), preferred_element_type=f32)
    t_m = _gelu(dot(hs, wm1a[...]) + dot(ee, wm1b[...]) + bm1[...])
    msg = dot(t_m, wm2[...]) + bm2[...]
    t_g = _gelu(dot(hd, wg1d[...]) + dot(hs, wg1s[...]) + dot(ee, wg1e[...])
                + bg1[...])
    g16 = jax.nn.sigmoid(dot(t_g, wg2p[...]) + bg2p[...])
    t_a = _gelu(dot(hd, wa1d[...]) + dot(hs, wa1s[...]) + dot(ee, wa1e[...])
                + ba1[...])
    s16 = (dot(t_a, wa2p[...]) + ba2p[...]) * SCALE
    ex16 = jnp.exp(s16) * mask16[...]
    r = r_ref[...]
    u_ref[...] = msg * dot(g16, r) * dot(ex16, r)
    ex_ref[...] = ex16


def _edge_mlp(hs, hd, ee, wm1a, wm1b, wm2, wg1d, wg1s, wg1e, wg2p,
              wa1d, wa1s, wa1e, wa2p, rmat,
              bm1, bm2, bg1, bg2p, ba1, ba2p, mask16):
    full = lambda s: pl.BlockSpec(s, lambda i: tuple(0 for _ in s))
    return pl.pallas_call(
        _edge_body,
        grid=(E // BE,),
        in_specs=[
            pl.BlockSpec((BE, D), lambda i: (i, 0)),
            pl.BlockSpec((BE, D), lambda i: (i, 0)),
            pl.BlockSpec((BE, DE), lambda i: (i, 0)),
            full((D, D)), full((DE, D)), full((D, D)),
            full((D, D)), full((D, D)), full((DE, D)), full((D, 16)),
            full((D, D)), full((D, D)), full((DE, D)), full((D, 16)),
            full((16, D)),
            full((1, D)), full((1, D)), full((1, D)), full((1, 16)),
            full((1, D)), full((1, 16)), full((1, 16)),
        ],
        out_specs=[
            pl.BlockSpec((BE, D), lambda i: (i, 0)),
            pl.BlockSpec((BE, 16), lambda i: (i, 0)),
        ],
        out_shape=[
            jax.ShapeDtypeStruct((E, D), jnp.float32),
            jax.ShapeDtypeStruct((E, 16), jnp.float32),
        ],
    )(hs, hd, ee, wm1a, wm1b, wm2, wg1d, wg1s, wg1e, wg2p,
      wa1d, wa1s, wa1e, wa2p, rmat, bm1, bm2, bg1, bg2p, ba1, ba2p, mask16)


def _final_body(x_ref, h_ref, ap_ref, rdn_ref,
                wself, wagg, wf1, wf2,
                bsa, bf1, bf2, g2, b2, o_ref):
    f32 = jnp.float32
    dot = lambda a, b: jax.lax.dot_general(
        a, b, (((1,), (0,)), ((), ())), preferred_element_type=f32)
    agg = (ap_ref[0] + ap_ref[1]) * rdn_ref[...]
    upd = dot(h_ref[...], wself[...]) + dot(agg, wagg[...]) + bsa[...]
    o1 = x_ref[...] + upd
    m = jnp.mean(o1, axis=-1, keepdims=True)
    v = jnp.mean((o1 - m) * (o1 - m), axis=-1, keepdims=True)
    ln = (o1 - m) / jnp.sqrt(v + 1e-5) * g2[...] + b2[...]
    f = dot(_gelu(dot(ln, wf1[...]) + bf1[...]), wf2[...]) + bf2[...]
    o_ref[...] = o1 + f


def _final(x, h, agg_p, rdn_n, wself, wagg, wf1, wf2,
           bsa, bf1, bf2, g2, b2):
    full = lambda s: pl.BlockSpec(s, lambda i: tuple(0 for _ in s))
    return pl.pallas_call(
        _final_body,
        grid=(N // BN,),
        in_specs=[
            pl.BlockSpec((BN, D), lambda i: (i, 0)),
            pl.BlockSpec((BN, D), lambda i: (i, 0)),
            pl.BlockSpec((2, BN, D), lambda i: (0, i, 0)),
            pl.BlockSpec((BN, D), lambda i: (i, 0)),
            full((D, D)), full((D, D)), full((D, 2 * D)), full((2 * D, D)),
            full((1, D)), full((1, 2 * D)), full((1, D)),
            full((1, D)), full((1, D)),
        ],
        out_specs=pl.BlockSpec((BN, D), lambda i: (i, 0)),
        out_shape=jax.ShapeDtypeStruct((N, D), jnp.float32),
    )(x, h, agg_p, rdn_n, wself, wagg, wf1, wf2,
      bsa, bf1, bf2, g2, b2)


QB = 4


def _sc_gather_body(h_hbm, src3, dst3, hs_out, hd_out,
                    idx_s, idx_d, bufa, bufb, sema, semb):
    c = lax.axis_index("c")
    s = lax.axis_index("s")
    wid = c * NS + s
    base = wid * EPW
    pltpu.sync_copy(src3.at[wid], idx_s)
    pltpu.sync_copy(dst3.at[wid], idx_d)

    def _quad(q, _):
        j0 = q * QB
        cps = []
        for b in range(QB):
            cps.append(pltpu.async_copy(
                h_hbm.at[idx_s.at[j0 + b]],
                bufa.at[pl.ds(b * CH, CH), :], sema))
            cps.append(pltpu.async_copy(
                h_hbm.at[idx_d.at[j0 + b]],
                bufb.at[pl.ds(b * CH, CH), :], semb))
        for cp in cps:
            cp.wait()
        pltpu.sync_copy(bufa, hs_out.at[pl.ds(base + j0 * CH, QB * CH), :])
        pltpu.sync_copy(bufb, hd_out.at[pl.ds(base + j0 * CH, QB * CH), :])
        return 0
    lax.fori_loop(0, KJ // QB, _quad, 0)

    for j in range(KJ - KJ % QB, KJ):
        cpa = pltpu.async_copy(h_hbm.at[idx_s.at[j]],
                               bufa.at[pl.ds(0, CH), :], sema)
        cpb = pltpu.async_copy(h_hbm.at[idx_d.at[j]],
                               bufb.at[pl.ds(0, CH), :], semb)
        cpa.wait()
        pltpu.sync_copy(bufa.at[pl.ds(0, CH), :],
                        hs_out.at[pl.ds(base + j * CH, CH), :])
        cpb.wait()
        pltpu.sync_copy(bufb.at[pl.ds(0, CH), :],
                        hd_out.at[pl.ds(base + j * CH, CH), :])


def _sc_gather(h, src3, dst3):
    f32 = jnp.float32
    return pl.kernel(
        _sc_gather_body,
        out_type=[jax.ShapeDtypeStruct((E, D), f32),
                  jax.ShapeDtypeStruct((E, D), f32)],
        mesh=plsc.VectorSubcoreMesh(core_axis_name="c", subcore_axis_name="s"),
        scratch_types=[
            pltpu.VMEM((KJ, CH), jnp.int32),
            pltpu.VMEM((KJ, CH), jnp.int32),
            pltpu.VMEM((QB * CH, D), f32),
            pltpu.VMEM((QB * CH, D), f32),
            pltpu.SemaphoreType.DMA,
            pltpu.SemaphoreType.DMA,
        ],
    )(h, src3, dst3)


def _sc_agg_body(dst3, u_hbm, agg_out, idx_v, ub, agg_s):
    c = lax.axis_index("c")
    s = lax.axis_index("s")
    wid = c * NS + s
    base = wid * EPW
    z16 = jnp.zeros((16,), jnp.float32)

    def _zrow(r, _):
        for l in range(D // 16):
            ub[r, pl.ds(l * 16, 16)] = z16
        return 0
    lax.fori_loop(0, CH, _zrow, 0)

    def _zdma(t, _):
        pltpu.sync_copy(ub, agg_s.at[pl.ds(s * NPS + t * CH, CH), :])
        return 0
    lax.fori_loop(0, NPS // CH, _zdma, 0)
    plsc.subcore_barrier()

    pltpu.sync_copy(dst3.at[wid], idx_v)

    def _step(j, _):
        pltpu.sync_copy(u_hbm.at[pl.ds(base + j * CH, CH), :], ub)
        pltpu.sync_copy(ub, agg_s.at[idx_v.at[j]], add=True)
        return 0
    lax.fori_loop(0, KJ, _step, 0)
    plsc.subcore_barrier()

    pltpu.sync_copy(agg_s.at[pl.ds(s * NPS, NPS), :],
                    agg_out.at[c, pl.ds(s * NPS, NPS), :])


def _sc_agg(dst3, u):
    f32 = jnp.float32
    return pl.kernel(
        _sc_agg_body,
        out_type=jax.ShapeDtypeStruct((NC, NP, D), f32),
        mesh=plsc.VectorSubcoreMesh(core_axis_name="c", subcore_axis_name="s"),
        scratch_types=[
            pltpu.VMEM((KJ, CH), jnp.int32),
            pltpu.VMEM((CH, D), f32),
            pltpu.VMEM_SHARED((NP, D), f32),
        ],
    )(dst3, u)


def _sc_dnacc_body(dst3, ex_hbm, dn_out, idx_v, exb, pb, dn_s):
    c = lax.axis_index("c")
    s = lax.axis_index("s")
    wid = c * NS + s
    base = wid * EPW
    z16 = jnp.zeros((16,), jnp.float32)

    def _zrow(r, _):
        for l in range(D // 16):
            pb[r, pl.ds(l * 16, 16)] = z16
        return 0
    lax.fori_loop(0, CH, _zrow, 0)

    def _zdma(t, _):
        pltpu.sync_copy(pb, dn_s.at[pl.ds(s * NPS + t * CH, CH), :])
        return 0
    lax.fori_loop(0, NPS // CH, _zdma, 0)
    plsc.subcore_barrier()

    pltpu.sync_copy(dst3.at[wid], idx_v)

    def _step(j, _):
        pltpu.sync_copy(ex_hbm.at[pl.ds(base + j * CH, CH), :], exb)

        def _row(r, _):
            pb[r, pl.ds(0, 16)] = exb[r, :]
            return 0
        lax.fori_loop(0, CH, _row, 0)
        pltpu.sync_copy(pb, dn_s.at[idx_v.at[j]], add=True)
        return 0
    lax.fori_loop(0, KJ, _step, 0)
    plsc.subcore_barrier()

    pltpu.sync_copy(dn_s.at[pl.ds(s * NPS, NPS), :],
                    dn_out.at[c, pl.ds(s * NPS, NPS), :])


def _sc_dnacc(dst3, ex):
    f32 = jnp.float32
    return pl.kernel(
        _sc_dnacc_body,
        out_type=jax.ShapeDtypeStruct((NC, NP, D), f32),
        mesh=plsc.VectorSubcoreMesh(core_axis_name="c", subcore_axis_name="s"),
        scratch_types=[
            pltpu.VMEM((KJ, CH), jnp.int32),
            pltpu.VMEM((CH, 16), f32),
            pltpu.VMEM((CH, D), f32),
            pltpu.VMEM_SHARED((NP, D), f32),
        ],
    )(dst3, ex)


BNP = 1280


def _rdnrep_body(d_ref, r_ref, o_ref):
    dn16 = d_ref[0, :, :16] + d_ref[1, :, :16]
    rdn = 1.0 / jnp.clip(dn16, 1e-12, None)
    o_ref[...] = jax.lax.dot_general(
        rdn, r_ref[...], (((1,), (0,)), ((), ())),
        preferred_element_type=jnp.float32)


def _rdnrep(dn_pp, rmat):
    return pl.pallas_call(
        _rdnrep_body,
        grid=(NP // BNP,),
        in_specs=[pl.BlockSpec((NC, BNP, D), lambda i: (0, i, 0)),
                  pl.BlockSpec((16, D), lambda i: (0, 0))],
        out_specs=pl.BlockSpec((BNP, D), lambda i: (i, 0)),
        out_shape=jax.ShapeDtypeStruct((NP, D), jnp.float32),
    )(dn_pp, rmat)


def _sc_er_body(u_hbm, rdnrep, dst3, er_out, idx_v, ub, gb, rdn_s, sem):
    c = lax.axis_index("c")
    s = lax.axis_index("s")
    wid = c * NS + s
    base = wid * EPW

    @pl.when(s == 0)
    def _():
        pltpu.sync_copy(rdnrep, rdn_s)
    plsc.subcore_barrier()

    pltpu.sync_copy(dst3.at[wid], idx_v)

    def _step(j, _):
        pltpu.sync_copy(u_hbm.at[pl.ds(base + j * CH, CH), :], ub)
        pltpu.async_copy(rdn_s.at[idx_v.at[j]], gb, sem).wait()

        def _row(r, _):
            for l in range(D // 16):
                sl = pl.ds(l * 16, 16)
                ub[r, sl] = ub[r, sl] * gb[r, sl]
            return 0
        lax.fori_loop(0, CH, _row, 0)
        pltpu.sync_copy(ub, er_out.at[pl.ds(base + j * CH, CH), :])
        return 0
    lax.fori_loop(0, KJ, _step, 0)


def _sc_er(u, rdnrep, dst3):
    f32 = jnp.float32
    return pl.kernel(
        _sc_er_body,
        out_type=jax.ShapeDtypeStruct((E, D), f32),
        mesh=plsc.VectorSubcoreMesh(core_axis_name="c", subcore_axis_name="s"),
        scratch_types=[
            pltpu.VMEM((KJ, CH), jnp.int32),
            pltpu.VMEM((CH, D), f32),
            pltpu.VMEM((CH, D), f32),
            pltpu.VMEM_SHARED((NP, D), f32),
            pltpu.SemaphoreType.DMA,
        ],
    )(u, rdnrep, dst3)


def kernel(x, edge_src, edge_dst, edge_emb, ln1_g, ln1_b, ln2_g, ln2_b,
           W_self, b_self, W_msg1, b_msg1, W_msg2, b_msg2,
           W_attn1, b_attn1, W_attn2, b_attn2,
           W_gate1, b_gate1, W_gate2, b_gate2,
           W_agg, b_agg, W_ffn1, b_ffn1, W_ffn2, b_ffn2):
    f32 = jnp.float32
    wm1a, wm1b = W_msg1[:D], W_msg1[D:]
    wg1d, wg1s, wg1e = W_gate1[:D], W_gate1[D:2 * D], W_gate1[2 * D:]
    wa1d, wa1s, wa1e = W_attn1[:D], W_attn1[D:2 * D], W_attn1[2 * D:]
    wg2p = jnp.pad(W_gate2, ((0, 0), (0, 16 - H)))
    wa2p = jnp.pad(W_attn2, ((0, 0), (0, 16 - H)))
    bg2p = jnp.pad(b_gate2, (0, 16 - H))[None, :]
    ba2p = jnp.pad(b_attn2, (0, 16 - H))[None, :]
    rmat = (jnp.arange(D)[None, :] // HD == jnp.arange(16)[:, None]).astype(f32)
    mask16 = (jnp.arange(16) < H).astype(f32)[None, :]
    bsa = (b_self + b_agg)[None, :]

    h = _ln1(x, ln1_g, ln1_b)

    src3 = edge_src.reshape(NW, KJ, CH)
    dst3 = edge_dst.reshape(NW, KJ, CH)
    hs, hd = _sc_gather(h, src3, dst3)

    u, ex = _edge_mlp(hs, hd, edge_emb,
                      wm1a, wm1b, W_msg2, wg1d, wg1s, wg1e, wg2p,
                      wa1d, wa1s, wa1e, wa2p, rmat,
                      b_msg1[None, :], b_msg2[None, :], b_gate1[None, :],
                      bg2p, b_attn1[None, :], ba2p, mask16)

    agg_pp = _sc_agg(dst3, u)
    agg_p = agg_pp[:, :N]

    dn_pp = _sc_dnacc(dst3, ex)
    rdn_rep = _rdnrep(dn_pp, rmat)

    er = _sc_er(u, rdn_rep, dst3)

    out = _final(x, h, agg_p, rdn_rep[:N], W_self, W_agg, W_ffn1, W_ffn2,
                 bsa, b_ffn1[None, :], b_ffn2[None, :],
                 ln2_g[None, :], ln2_b[None, :])
    return (out, er)


# speedup vs baseline: 1.0218x; 1.0038x over previous
import functools

import jax
import jax.numpy as jnp
from jax import lax
from jax.experimental import pallas as pl
from jax.experimental.pallas import tpu as pltpu
from jax.experimental.pallas import tpu_sc as plsc

N = 10000
E = 320000
D = 128
DE = 16
H = 4
HD = D // H
SCALE = 1.0 / (HD ** 0.5)

BN = 1000
BE = 512

NC = 2
NS = 16
NW = NC * NS
EPW = E // NW
CH = 80
KJ = EPW // CH
NP = 10240
NPS = NP // NS
ZR = 128


def _gelu(t):
    return 0.5 * t * (1.0 + lax.erf(t * (2.0 ** -0.5)))


def _ln_body(x_ref, g_ref, b_ref, o_ref):
    x = x_ref[...]
    m = jnp.mean(x, axis=-1, keepdims=True)
    v = jnp.mean((x - m) * (x - m), axis=-1, keepdims=True)
    o_ref[...] = (x - m) / jnp.sqrt(v + 1e-5) * g_ref[...] + b_ref[...]


def _ln1(x, g, b):
    return pl.pallas_call(
        _ln_body,
        grid=(N // BN,),
        in_specs=[
            pl.BlockSpec((BN, D), lambda i: (i, 0)),
            pl.BlockSpec((1, D), lambda i: (0, 0)),
            pl.BlockSpec((1, D), lambda i: (0, 0)),
        ],
        out_specs=pl.BlockSpec((BN, D), lambda i: (i, 0)),
        out_shape=jax.ShapeDtypeStruct((N, D), jnp.float32),
    )(x, g[None, :], b[None, :])


def _edge_body(hs_ref, hd_ref, ee_ref,
               wm1a, wm1b, wm2, wg1d, wg1s, wg1e, wg2p,
               wa1d, wa1s, wa1e, wa2p, r_ref,
               bm1, bm2, bg1, bg2p, ba1, ba2p, mask16,
               u_ref, ex_ref):
    hs = hs_ref[...]
    hd = hd_ref[...]
    ee = ee_ref[...]
    f32 = jnp.float32
    dot = lambda a, b: jax.lax.dot_general(
        a, b, (((1,), (0,)), ((), ())), preferred_element_type=f32)
    t_m = _gelu(dot(hs, wm1a[...]) + dot(ee, wm1b[...]) + bm1[...])
    msg = dot(t_m, wm2[...]) + bm2[...]
    t_g = _gelu(dot(hd, wg1d[...]) + dot(hs, wg1s[...]) + dot(ee, wg1e[...])
                + bg1[...])
    g16 = jax.nn.sigmoid(dot(t_g, wg2p[...]) + bg2p[...])
    t_a = _gelu(dot(hd, wa1d[...]) + dot(hs, wa1s[...]) + dot(ee, wa1e[...])
                + ba1[...])
    s16 = (dot(t_a, wa2p[...]) + ba2p[...]) * SCALE
    ex16 = jnp.exp(s16) * mask16[...]
    r = r_ref[...]
    u_ref[...] = msg * dot(g16, r) * dot(ex16, r)
    ex_ref[...] = ex16


def _edge_mlp(hs, hd, ee, wm1a, wm1b, wm2, wg1d, wg1s, wg1e, wg2p,
              wa1d, wa1s, wa1e, wa2p, rmat,
              bm1, bm2, bg1, bg2p, ba1, ba2p, mask16):
    full = lambda s: pl.BlockSpec(s, lambda i: tuple(0 for _ in s))
    return pl.pallas_call(
        _edge_body,
        grid=(E // BE,),
        in_specs=[
            pl.BlockSpec((BE, D), lambda i: (i, 0)),
            pl.BlockSpec((BE, D), lambda i: (i, 0)),
            pl.BlockSpec((BE, DE), lambda i: (i, 0)),
            full((D, D)), full((DE, D)), full((D, D)),
            full((D, D)), full((D, D)), full((DE, D)), full((D, 16)),
            full((D, D)), full((D, D)), full((DE, D)), full((D, 16)),
            full((16, D)),
            full((1, D)), full((1, D)), full((1, D)), full((1, 16)),
            full((1, D)), full((1, 16)), full((1, 16)),
        ],
        out_specs=[
            pl.BlockSpec((BE, D), lambda i: (i, 0)),
            pl.BlockSpec((BE, 16), lambda i: (i, 0)),
        ],
        out_shape=[
            jax.ShapeDtypeStruct((E, D), jnp.float32),
            jax.ShapeDtypeStruct((E, 16), jnp.float32),
        ],
    )(hs, hd, ee, wm1a, wm1b, wm2, wg1d, wg1s, wg1e, wg2p,
      wa1d, wa1s, wa1e, wa2p, rmat, bm1, bm2, bg1, bg2p, ba1, ba2p, mask16)


def _final_body(x_ref, h_ref, ap_ref, rdn_ref,
                wself, wagg, wf1, wf2,
                bsa, bf1, bf2, g2, b2, o_ref):
    f32 = jnp.float32
    dot = lambda a, b: jax.lax.dot_general(
        a, b, (((1,), (0,)), ((), ())), preferred_element_type=f32)
    agg = (ap_ref[0] + ap_ref[1]) * rdn_ref[...]
    upd = dot(h_ref[...], wself[...]) + dot(agg, wagg[...]) + bsa[...]
    o1 = x_ref[...] + upd
    m = jnp.mean(o1, axis=-1, keepdims=True)
    v = jnp.mean((o1 - m) * (o1 - m), axis=-1, keepdims=True)
    ln = (o1 - m) / jnp.sqrt(v + 1e-5) * g2[...] + b2[...]
    f = dot(_gelu(dot(ln, wf1[...]) + bf1[...]), wf2[...]) + bf2[...]
    o_ref[...] = o1 + f


def _final(x, h, agg_p, rdn_n, wself, wagg, wf1, wf2,
           bsa, bf1, bf2, g2, b2):
    full = lambda s: pl.BlockSpec(s, lambda i: tuple(0 for _ in s))
    return pl.pallas_call(
        _final_body,
        grid=(N // BN,),
        in_specs=[
            pl.BlockSpec((BN, D), lambda i: (i, 0)),
            pl.BlockSpec((BN, D), lambda i: (i, 0)),
            pl.BlockSpec((2, BN, D), lambda i: (0, i, 0)),
            pl.BlockSpec((BN, D), lambda i: (i, 0)),
            full((D, D)), full((D, D)), full((D, 2 * D)), full((2 * D, D)),
            full((1, D)), full((1, 2 * D)), full((1, D)),
            full((1, D)), full((1, D)),
        ],
        out_specs=pl.BlockSpec((BN, D), lambda i: (i, 0)),
        out_shape=jax.ShapeDtypeStruct((N, D), jnp.float32),
    )(x, h, agg_p, rdn_n, wself, wagg, wf1, wf2,
      bsa, bf1, bf2, g2, b2)


QB = 4


def _sc_gather_body(h_hbm, src3, dst3, hs_out, hd_out,
                    idx_s, idx_d, bufa, bufb, sema, semb):
    c = lax.axis_index("c")
    s = lax.axis_index("s")
    wid = c * NS + s
    base = wid * EPW
    pltpu.sync_copy(src3.at[wid], idx_s)
    pltpu.sync_copy(dst3.at[wid], idx_d)

    def _quad(q, _):
        j0 = q * QB
        cps = []
        for b in range(QB):
            cps.append(pltpu.async_copy(
                h_hbm.at[idx_s.at[j0 + b]],
                bufa.at[pl.ds(b * CH, CH), :], sema))
            cps.append(pltpu.async_copy(
                h_hbm.at[idx_d.at[j0 + b]],
                bufb.at[pl.ds(b * CH, CH), :], semb))
        for cp in cps:
            cp.wait()
        pltpu.sync_copy(bufa, hs_out.at[pl.ds(base + j0 * CH, QB * CH), :])
        pltpu.sync_copy(bufb, hd_out.at[pl.ds(base + j0 * CH, QB * CH), :])
        return 0
    lax.fori_loop(0, KJ // QB, _quad, 0)

    for j in range(KJ - KJ % QB, KJ):
        cpa = pltpu.async_copy(h_hbm.at[idx_s.at[j]],
                               bufa.at[pl.ds(0, CH), :], sema)
        cpb = pltpu.async_copy(h_hbm.at[idx_d.at[j]],
                               bufb.at[pl.ds(0, CH), :], semb)
        cpa.wait()
        pltpu.sync_copy(bufa.at[pl.ds(0, CH), :],
                        hs_out.at[pl.ds(base + j * CH, CH), :])
        cpb.wait()
        pltpu.sync_copy(bufb.at[pl.ds(0, CH), :],
                        hd_out.at[pl.ds(base + j * CH, CH), :])


def _sc_gather(h, src3, dst3):
    f32 = jnp.float32
    return pl.kernel(
        _sc_gather_body,
        out_type=[jax.ShapeDtypeStruct((E, D), f32),
                  jax.ShapeDtypeStruct((E, D), f32)],
        mesh=plsc.VectorSubcoreMesh(core_axis_name="c", subcore_axis_name="s"),
        scratch_types=[
            pltpu.VMEM((KJ, CH), jnp.int32),
            pltpu.VMEM((KJ, CH), jnp.int32),
            pltpu.VMEM((QB * CH, D), f32),
            pltpu.VMEM((QB * CH, D), f32),
            pltpu.SemaphoreType.DMA,
            pltpu.SemaphoreType.DMA,
        ],
    )(h, src3, dst3)


def _sc_agg_body(dst3, u_hbm, agg_out, idx_v, ub, agg_s):
    c = lax.axis_index("c")
    s = lax.axis_index("s")
    wid = c * NS + s
    base = wid * EPW
    z16 = jnp.zeros((16,), jnp.float32)

    def _zrow(r, _):
        for l in range(D // 16):
            ub[r, pl.ds(l * 16, 16)] = z16
        return 0
    lax.fori_loop(0, CH, _zrow, 0)

    def _zdma(t, _):
        pltpu.sync_copy(ub, agg_s.at[pl.ds(s * NPS + t * CH, CH), :])
        return 0
    lax.fori_loop(0, NPS // CH, _zdma, 0)
    plsc.subcore_barrier()

    pltpu.sync_copy(dst3.at[wid], idx_v)

    def _step(j, _):
        pltpu.sync_copy(u_hbm.at[pl.ds(base + j * CH, CH), :], ub)
        pltpu.sync_copy(ub, agg_s.at[idx_v.at[j]], add=True)
        return 0
    lax.fori_loop(0, KJ, _step, 0)
    plsc.subcore_barrier()

    pltpu.sync_copy(agg_s.at[pl.ds(s * NPS, NPS), :],
                    agg_out.at[c, pl.ds(s * NPS, NPS), :])


def _sc_agg(dst3, u):
    f32 = jnp.float32
    return pl.kernel(
        _sc_agg_body,
        out_type=jax.ShapeDtypeStruct((NC, NP, D), f32),
        mesh=plsc.VectorSubcoreMesh(core_axis_name="c", subcore_axis_name="s"),
        scratch_types=[
            pltpu.VMEM((KJ, CH), jnp.int32),
            pltpu.VMEM((CH, D), f32),
            pltpu.VMEM_SHARED((NP, D), f32),
        ],
    )(dst3, u)


def _sc_dnacc_body(dst3, ex_hbm, dn_out, idx_v, exb, pb, dn_s):
    c = lax.axis_index("c")
    s = lax.axis_index("s")
    wid = c * NS + s
    base = wid * EPW
    z16 = jnp.zeros((16,), jnp.float32)

    def _zrow(r, _):
        for l in range(D // 16):
            pb[r, pl.ds(l * 16, 16)] = z16
        return 0
    lax.fori_loop(0, CH, _zrow, 0)

    def _zdma(t, _):
        pltpu.sync_copy(pb, dn_s.at[pl.ds(s * NPS + t * CH, CH), :])
        return 0
    lax.fori_loop(0, NPS // CH, _zdma, 0)
    plsc.subcore_barrier()

    pltpu.sync_copy(dst3.at[wid], idx_v)

    def _step(j, _):
        pltpu.sync_copy(ex_hbm.at[pl.ds(base + j * CH, CH), :], exb)

        def _row(r, _):
            pb[r, pl.ds(0, 16)] = exb[r, :]
            return 0
        lax.fori_loop(0, CH, _row, 0)
        pltpu.sync_copy(pb, dn_s.at[idx_v.at[j]], add=True)
        return 0
    lax.fori_loop(0, KJ, _step, 0)
    plsc.subcore_barrier()

    pltpu.sync_copy(dn_s.at[pl.ds(s * NPS, NPS), :],
                    dn_out.at[c, pl.ds(s * NPS, NPS), :])


def _sc_dnacc(dst3, ex):
    f32 = jnp.float32
    return pl.kernel(
        _sc_dnacc_body,
        out_type=jax.ShapeDtypeStruct((NC, NP, D), f32),
        mesh=plsc.VectorSubcoreMesh(core_axis_name="c", subcore_axis_name="s"),
        scratch_types=[
            pltpu.VMEM((KJ, CH), jnp.int32),
            pltpu.VMEM((CH, 16), f32),
            pltpu.VMEM((CH, D), f32),
            pltpu.VMEM_SHARED((NP, D), f32),
        ],
    )(dst3, ex)


BNP = 1280


def _rdnrep_body(d_ref, r_ref, o_ref):
    dn16 = d_ref[0, :, :16] + d_ref[1, :, :16]
    rdn = 1.0 / jnp.clip(dn16, 1e-12, None)
    o_ref[...] = jax.lax.dot_general(
        rdn, r_ref[...], (((1,), (0,)), ((), ())),
        preferred_element_type=jnp.float32)


def _rdnrep(dn_pp, rmat):
    return pl.pallas_call(
        _rdnrep_body,
        grid=(NP // BNP,),
        in_specs=[pl.BlockSpec((NC, BNP, D), lambda i: (0, i, 0)),
                  pl.BlockSpec((16, D), lambda i: (0, 0))],
        out_specs=pl.BlockSpec((BNP, D), lambda i: (i, 0)),
        out_shape=jax.ShapeDtypeStruct((NP, D), jnp.float32),
    )(dn_pp, rmat)


def _sc_er_body(u_hbm, rdnrep, dst3, er_out, idx_v, ub, gb, rdn_s, sem):
    c = lax.axis_index("c")
    s = lax.axis_index("s")
    wid = c * NS + s
    base = wid * EPW

    @pl.when(s == 0)
    def _():
        pltpu.sync_copy(rdnrep, rdn_s)
    plsc.subcore_barrier()

    pltpu.sync_copy(dst3.at[wid], idx_v)

    def _step(j, _):
        pltpu.sync_copy(u_hbm.at[pl.ds(base + j * CH, CH), :], ub)
        pltpu.async_copy(rdn_s.at[idx_v.at[j]], gb, sem).wait()

        def _row(r, _):
            for l in range(D // 16):
                sl = pl.ds(l * 16, 16)
                ub[r, sl] = ub[r, sl] * gb[r, sl]
            return 0
        lax.fori_loop(0, CH, _row, 0)
        pltpu.sync_copy(ub, er_out.at[pl.ds(base + j * CH, CH), :])
        return 0
    lax.fori_loop(0, KJ, _step, 0)


def _sc_er(u, rdnrep, dst3):
    f32 = jnp.float32
    return pl.kernel(
        _sc_er_body,
        out_type=jax.ShapeDtypeStruct((E, D), f32),
        mesh=plsc.VectorSubcoreMesh(core_axis_name="c", subcore_axis_name="s"),
        scratch_types=[
            pltpu.VMEM((KJ, CH), jnp.int32),
            pltpu.VMEM((CH, D), f32),
            pltpu.VMEM((CH, D), f32),
            pltpu.VMEM_SHARED((NP, D), f32),
            pltpu.SemaphoreType.DMA,
        ],
    )(u, rdnrep, dst3)


def kernel(x, edge_src, edge_dst, edge_emb, ln1_g, ln1_b, ln2_g, ln2_b,
           W_self, b_self, W_msg1, b_msg1, W_msg2, b_msg2,
           W_attn1, b_attn1, W_attn2, b_attn2,
           W_gate1, b_gate1, W_gate2, b_gate2,
           W_agg, b_agg, W_ffn1, b_ffn1, W_ffn2, b_ffn2):
    f32 = jnp.float32
    wm1a, wm1b = W_msg1[:D], W_msg1[D:]
    wg1d, wg1s, wg1e = W_gate1[:D], W_gate1[D:2 * D], W_gate1[2 * D:]
    wa1d, wa1s, wa1e = W_attn1[:D], W_attn1[D:2 * D], W_attn1[2 * D:]
    wg2p = jnp.pad(W_gate2, ((0, 0), (0, 16 - H)))
    wa2p = jnp.pad(W_attn2, ((0, 0), (0, 16 - H)))
    bg2p = jnp.pad(b_gate2, (0, 16 - H))[None, :]
    ba2p = jnp.pad(b_attn2, (0, 16 - H))[None, :]
    rmat = (jnp.arange(D)[None, :] // HD == jnp.arange(16)[:, None]).astype(f32)
    mask16 = (jnp.arange(16) < H).astype(f32)[None, :]
    bsa = (b_self + b_agg)[None, :]

    h = _ln1(x, ln1_g, ln1_b)

    src3 = edge_src.reshape(NW, KJ, CH)
    dst3 = edge_dst.reshape(NW, KJ, CH)
    hs, hd = _sc_gather(h, src3, dst3)

    u, ex = _edge_mlp(hs, hd, edge_emb,
                      wm1a, wm1b, W_msg2, wg1d, wg1s, wg1e, wg2p,
                      wa1d, wa1s, wa1e, wa2p, rmat,
                      b_msg1[None, :], b_msg2[None, :], b_gate1[None, :],
                      bg2p, b_attn1[None, :], ba2p, mask16)

    agg_pp = _sc_agg(dst3, u)

    dn_pp = _sc_dnacc(dst3, ex)
    rdn_rep = _rdnrep(dn_pp, rmat)

    er = _sc_er(u, rdn_rep, dst3)

    out = _final(x, h, agg_pp, rdn_rep, W_self, W_agg, W_ffn1, W_ffn2,
                 bsa, b_ffn1[None, :], b_ffn2[None, :],
                 ln2_g[None, :], ln2_b[None, :])
    return (out, er)
